# baseline (device time: 100212 ns/iter reference)
import jax
import jax.numpy as jnp
from jax import lax
from jax.experimental import pallas as pl
from jax.experimental.pallas import tpu as pltpu

N_DEV = 16
NR = N_DEV // 2
NL = N_DEV // 2


def kernel(x, w_mat, scale_x, scale_w):
    m_per, k = x.shape
    _, n_per = w_mat.shape

    def body(x_ref, w_ref, sx_ref, sw_ref, out_ref,
             commr_ref, comml_ref, sendr_sems, recvr_sems,
             sendl_sems, recvl_sems):
        my = lax.axis_index("i")
        left = lax.rem(my - 1 + N_DEV, N_DEV)
        right = lax.rem(my + 1, N_DEV)

        barrier_sem = pltpu.get_barrier_semaphore()
        for nbr in (left, right):
            pl.semaphore_signal(
                barrier_sem, inc=1,
                device_id=(nbr,), device_id_type=pl.DeviceIdType.MESH,
            )
        pl.semaphore_wait(barrier_sem, 2)

        scale = sx_ref[0] * sw_ref[0]

        def gemm_store(origin, chunk):
            acc = jnp.dot(chunk.astype(jnp.bfloat16),
                          w_ref[:, :].astype(jnp.bfloat16),
                          preferred_element_type=jnp.float32)
            y = acc * scale
            out_ref[pl.ds(origin * m_per, m_per), :] = y * jax.nn.sigmoid(y)

        NP = 4
        half = m_per // 2
        qtr = m_per // NP

        def pieces_r(h):
            return (0, 1) if h == NR - 1 else (0, 1, 2, 3)

        def pieces_l(h):
            return (3, 2) if h == NL - 1 else (3, 2, 1, 0)

        def rdma_r(h, p):
            src = commr_ref.at[h, pl.ds(p * qtr, qtr)]
            dst = commr_ref.at[h + 1, pl.ds(p * qtr, qtr)]
            return pltpu.make_async_remote_copy(
                src_ref=src, dst_ref=dst,
                send_sem=sendr_sems.at[h, p], recv_sem=recvr_sems.at[h, p],
                device_id=(right,), device_id_type=pl.DeviceIdType.MESH,
            )

        def rdma_l(h, p):
            src_slot = commr_ref.at[0] if h == 0 else comml_ref.at[h]
            src = src_slot.at[pl.ds(p * qtr, qtr)]
            dst = comml_ref.at[h + 1, pl.ds(p * qtr, qtr)]
            return pltpu.make_async_remote_copy(
                src_ref=src, dst_ref=dst,
                send_sem=sendl_sems.at[h, p], recv_sem=recvl_sems.at[h, p],
                device_id=(left,), device_id_type=pl.DeviceIdType.MESH,
            )

        for p in (0, 3, 1, 2):
            commr_ref[0, pl.ds(p * qtr, qtr), :] = (
                x_ref[pl.ds(p * qtr, qtr), :].astype(jnp.float8_e4m3fn))
            if p in (0, 1):
                rdma_r(0, p).start()
            else:
                rdma_l(0, p).start()
        rdma_r(0, 2).start()
        rdma_r(0, 3).start()
        rdma_l(0, 1).start()
        rdma_l(0, 0).start()
        gemm_store(my, x_ref[:, :])

        for h in range(NR):
            for p in pieces_r(h):
                rdma_r(h, p).wait_recv()
                if h + 1 < NR and p in pieces_r(h + 1):
                    rdma_r(h + 1, p).start()
            for p in pieces_l(h):
                rdma_l(h, p).wait_recv()
                if h + 1 < NL and p in pieces_l(h + 1):
                    rdma_l(h + 1, p).start()
            if h == NR - 1:
                origin = lax.rem(my + NR, N_DEV)
                acc_t = jnp.dot(commr_ref[h + 1, :half, :].astype(jnp.bfloat16),
                                w_ref[:, :].astype(jnp.bfloat16),
                                preferred_element_type=jnp.float32)
                y_t = acc_t * scale
                out_ref[pl.ds(origin * m_per, half), :] = y_t * jax.nn.sigmoid(y_t)
                acc_b = jnp.dot(comml_ref[h + 1, half:, :].astype(jnp.bfloat16),
                                w_ref[:, :].astype(jnp.bfloat16),
                                preferred_element_type=jnp.float32)
                y_b = acc_b * scale
                out_ref[pl.ds(origin * m_per + half, half), :] = (
                    y_b * jax.nn.sigmoid(y_b))
            else:
                gemm_store(lax.rem(my - h - 1 + N_DEV, N_DEV),
                           commr_ref[h + 1, :, :])
                gemm_store(lax.rem(my + h + 1, N_DEV),
                           comml_ref[h + 1, :, :])

        for h in range(NR):
            for p in pieces_r(h):
                rdma_r(h, p).wait_send()
        for h in range(NL):
            for p in pieces_l(h):
                rdma_l(h, p).wait_send()

    return pl.pallas_call(
        body,
        out_shape=jax.ShapeDtypeStruct((N_DEV * m_per, n_per), jnp.float32),
        in_specs=[
            pl.BlockSpec(memory_space=pltpu.VMEM),
            pl.BlockSpec(memory_space=pltpu.VMEM),
            pl.BlockSpec(memory_space=pltpu.SMEM),
            pl.BlockSpec(memory_space=pltpu.SMEM),
        ],
        out_specs=pl.BlockSpec(memory_space=pltpu.VMEM),
        scratch_shapes=[
            pltpu.VMEM((NR + 1, m_per, k), jnp.float8_e4m3fn),
            pltpu.VMEM((NL + 1, m_per, k), jnp.float8_e4m3fn),
            pltpu.SemaphoreType.DMA((NR, 4)),
            pltpu.SemaphoreType.DMA((NR, 4)),
            pltpu.SemaphoreType.DMA((NL, 4)),
            pltpu.SemaphoreType.DMA((NL, 4)),
        ],
        compiler_params=pltpu.CompilerParams(collective_id=0),
    )(x, w_mat, scale_x, scale_w)


# device time: 99621 ns/iter; 1.0059x vs baseline; 1.0059x over previous
import jax
import jax.numpy as jnp
from jax import lax
from jax.experimental import pallas as pl
from jax.experimental.pallas import tpu as pltpu

N_DEV = 16
NR = N_DEV // 2
NL = N_DEV // 2


def kernel(x, w_mat, scale_x, scale_w):
    m_per, k = x.shape
    _, n_per = w_mat.shape

    def body(x_ref, w_ref, sx_ref, sw_ref, out_ref,
             commr_ref, comml_ref, sendr_sems, recvr_sems,
             sendl_sems, recvl_sems):
        my = lax.axis_index("i")
        left = lax.rem(my - 1 + N_DEV, N_DEV)
        right = lax.rem(my + 1, N_DEV)

        barrier_sem = pltpu.get_barrier_semaphore()
        for nbr in (left, right):
            pl.semaphore_signal(
                barrier_sem, inc=1,
                device_id=(nbr,), device_id_type=pl.DeviceIdType.MESH,
            )
        pl.semaphore_wait(barrier_sem, 2)

        scale = sx_ref[0] * sw_ref[0]

        def gemm_store(origin, chunk):
            acc = jnp.dot(chunk.astype(jnp.bfloat16),
                          w_ref[:, :].astype(jnp.bfloat16),
                          preferred_element_type=jnp.float32)
            y = acc * scale
            out_ref[pl.ds(origin * m_per, m_per), :] = y * jax.nn.sigmoid(y)

        half = m_per // 2

        def pieces_r(h):
            return (0,) if h == NR - 1 else (0, 1)

        def pieces_l(h):
            return (1,) if h == NL - 1 else (1, 0)

        def rdma_r(h, p):
            src = commr_ref.at[h, pl.ds(p * half, half)]
            dst = commr_ref.at[h + 1, pl.ds(p * half, half)]
            return pltpu.make_async_remote_copy(
                src_ref=src, dst_ref=dst,
                send_sem=sendr_sems.at[h, p], recv_sem=recvr_sems.at[h, p],
                device_id=(right,), device_id_type=pl.DeviceIdType.MESH,
            )

        def rdma_l(h, p):
            src_slot = commr_ref.at[0] if h == 0 else comml_ref.at[h]
            src = src_slot.at[pl.ds(p * half, half)]
            dst = comml_ref.at[h + 1, pl.ds(p * half, half)]
            return pltpu.make_async_remote_copy(
                src_ref=src, dst_ref=dst,
                send_sem=sendl_sems.at[h, p], recv_sem=recvl_sems.at[h, p],
                device_id=(left,), device_id_type=pl.DeviceIdType.MESH,
            )

        commr_ref[0, pl.ds(0, half), :] = (
            x_ref[pl.ds(0, half), :].astype(jnp.float8_e4m3fn))
        rdma_r(0, 0).start()
        commr_ref[0, pl.ds(half, half), :] = (
            x_ref[pl.ds(half, half), :].astype(jnp.float8_e4m3fn))
        rdma_l(0, 1).start()
        rdma_r(0, 1).start()
        rdma_l(0, 0).start()
        gemm_store(my, x_ref[:, :])

        for h in range(NR):
            for p in pieces_r(h):
                rdma_r(h, p).wait_recv()
                if h + 1 < NR and p in pieces_r(h + 1):
                    rdma_r(h + 1, p).start()
            for p in pieces_l(h):
                rdma_l(h, p).wait_recv()
                if h + 1 < NL and p in pieces_l(h + 1):
                    rdma_l(h + 1, p).start()
            if h == NR - 1:
                origin = lax.rem(my + NR, N_DEV)
                acc_t = jnp.dot(commr_ref[h + 1, :half, :].astype(jnp.bfloat16),
                                w_ref[:, :].astype(jnp.bfloat16),
                                preferred_element_type=jnp.float32)
                y_t = acc_t * scale
                out_ref[pl.ds(origin * m_per, half), :] = y_t * jax.nn.sigmoid(y_t)
                acc_b = jnp.dot(comml_ref[h + 1, half:, :].astype(jnp.bfloat16),
                                w_ref[:, :].astype(jnp.bfloat16),
                                preferred_element_type=jnp.float32)
                y_b = acc_b * scale
                out_ref[pl.ds(origin * m_per + half, half), :] = (
                    y_b * jax.nn.sigmoid(y_b))
            else:
                gemm_store(lax.rem(my - h - 1 + N_DEV, N_DEV),
                           commr_ref[h + 1, :, :])
                gemm_store(lax.rem(my + h + 1, N_DEV),
                           comml_ref[h + 1, :, :])

        for h in range(NR):
            for p in pieces_r(h):
                rdma_r(h, p).wait_send()
        for h in range(NL):
            for p in pieces_l(h):
                rdma_l(h, p).wait_send()

    return pl.pallas_call(
        body,
        out_shape=jax.ShapeDtypeStruct((N_DEV * m_per, n_per), jnp.float32),
        in_specs=[
            pl.BlockSpec(memory_space=pltpu.VMEM),
            pl.BlockSpec(memory_space=pltpu.VMEM),
            pl.BlockSpec(memory_space=pltpu.SMEM),
            pl.BlockSpec(memory_space=pltpu.SMEM),
        ],
        out_specs=pl.BlockSpec(memory_space=pltpu.VMEM),
        scratch_shapes=[
            pltpu.VMEM((NR + 1, m_per, k), jnp.float8_e4m3fn),
            pltpu.VMEM((NL + 1, m_per, k), jnp.float8_e4m3fn),
            pltpu.SemaphoreType.DMA((NR, 2)),
            pltpu.SemaphoreType.DMA((NR, 2)),
            pltpu.SemaphoreType.DMA((NL, 2)),
            pltpu.SemaphoreType.DMA((NL, 2)),
        ],
        compiler_params=pltpu.CompilerParams(collective_id=0),
    )(x, w_mat, scale_x, scale_w)
